# baseline (speedup 1.0000x reference)
import functools

import jax
import jax.numpy as jnp
from jax import lax
from jax.experimental import pallas as pl
from jax.experimental.pallas import tpu as pltpu
from jax.experimental.pallas import tpu_sc as plsc

B = 16384
N_CAT = 26
N_NUM = 8
VOCAB = 100000
CHANNELS = 16
N_COLS = N_CAT + N_NUM
INV = 1.0 / N_COLS

NC = 2
NS = 16
NW = NC * NS
ROWS_PER_W = B // NW
CHUNK = 128
CHUNKS_PER_W = ROWS_PER_W // CHUNK
IDX_PER_CHUNK = CHUNK * N_CAT
IDX_GRP = 128
N_GRP = IDX_PER_CHUNK // IDX_GRP


def _base_body(num_ref, w_ref, b_ref, out_ref):
    b_sum = jnp.sum(b_ref[...], axis=0, keepdims=True)
    out_ref[...] = (
        jnp.dot(num_ref[...], w_ref[...], preferred_element_type=jnp.float32)
        + b_sum
    ) * INV


def _base(num_feat, lin_w, lin_b):
    return pl.pallas_call(
        _base_body,
        out_shape=jax.ShapeDtypeStruct((B, CHANNELS), jnp.float32),
    )(num_feat, lin_w, lin_b)


@functools.partial(
    pl.kernel,
    out_type=jax.ShapeDtypeStruct((B, CHANNELS), jnp.float32),
    mesh=plsc.VectorSubcoreMesh(core_axis_name="c", subcore_axis_name="s"),
    compiler_params=pltpu.CompilerParams(use_tc_tiling_on_sc=False),
    scratch_types=[
        pltpu.VMEM((ROWS_PER_W * N_CAT // IDX_GRP, IDX_GRP), jnp.int32),
        pltpu.VMEM((IDX_PER_CHUNK, CHANNELS), jnp.float32),
        pltpu.VMEM((CHUNK, CHANNELS), jnp.float32),
        pltpu.VMEM((CHUNK, CHANNELS), jnp.float32),
        pltpu.SemaphoreType.DMA,
    ],
)
def _sc_gather(table, idx2d, base, out, idx_v, rows_v, base_v, out_v, sem):
    wid = lax.axis_index("s") * NC + lax.axis_index("c")
    w_irows = ROWS_PER_W * N_CAT // IDX_GRP
    pltpu.sync_copy(idx2d.at[pl.ds(wid * w_irows, w_irows)], idx_v)

    def chunk_body(c, carry):
        row0 = (wid * CHUNKS_PER_W + c) * CHUNK
        descs = []
        for g in range(N_GRP):
            descs.append(
                pltpu.async_copy(
                    table.at[idx_v.at[c * N_GRP + g]],
                    rows_v.at[pl.ds(g * IDX_GRP, IDX_GRP)],
                    sem,
                )
            )
        pltpu.sync_copy(base.at[pl.ds(row0, CHUNK)], base_v)
        for d in descs:
            d.wait()

        def row_body(r, rcarry):
            p0 = r * N_CAT
            acc = rows_v[p0]
            for j in range(1, N_CAT):
                acc = acc + rows_v[p0 + j]
            out_v[r] = base_v[r] + acc * INV
            return rcarry

        lax.fori_loop(0, CHUNK, row_body, 0)
        pltpu.sync_copy(out_v, out.at[pl.ds(row0, CHUNK)])
        return carry

    lax.fori_loop(0, CHUNKS_PER_W, chunk_body, 0)


def kernel(cat_idx, num_feat, emb_tables, lin_w, lin_b):
    base = _base(num_feat, lin_w, lin_b)
    offs = (jnp.arange(N_CAT, dtype=jnp.int32) * VOCAB)[None, :]
    flat_idx = (cat_idx.astype(jnp.int32) + offs).reshape(
        B * N_CAT // IDX_GRP, IDX_GRP
    )
    table = emb_tables.reshape(N_CAT * VOCAB, CHANNELS)
    return _sc_gather(table, flat_idx, base)
